# baseline (speedup 1.0000x reference)
import functools

import jax
import jax.numpy as jnp
from jax import lax
from jax.experimental import pallas as pl
from jax.experimental.pallas import tpu as pltpu
from jax.experimental.pallas import tpu_sc as plsc

B = 16384
V = 1000000
D = 64
NEG = 20

NC = 2
NS = 16
L = 16
NW = NC * NS
BW = B // NW
C = 32
NCHUNK = BW // C
GC = C // L
IDXBLK = 128


def _sc_score_body(cid_hbm, pid_hbm, nid_hbm, cw_hbm, xw_hbm,
                   pos_out, neg_out,
                   cid_v, pid_v, nid_v, crow, prow, nrow,
                   psc, nsc, gsem):
    c = lax.axis_index("c")
    s = lax.axis_index("s")
    wid = s * NC + c
    base = wid * BW

    pltpu.sync_copy(cid_hbm.at[pl.ds(base, BW)], cid_v)
    pltpu.sync_copy(pid_hbm.at[pl.ds(base, BW)], pid_v)
    pltpu.sync_copy(nid_hbm.at[pl.ds(base * NEG, BW * NEG)], nid_v)

    lanes = lax.iota(jnp.int32, L)
    cols = [jnp.full((L,), d, jnp.int32) for d in range(D)]

    def chunk_body(g, carry):
        cb = g * C
        copies = [
            pltpu.async_copy(cw_hbm.at[cid_v.at[pl.ds(cb, C)]],
                             crow, gsem),
            pltpu.async_copy(xw_hbm.at[pid_v.at[pl.ds(cb, C)]],
                             prow, gsem),
        ]
        nrow2 = nrow
        for i in range(C * NEG // IDXBLK):
            copies.append(pltpu.async_copy(
                xw_hbm.at[nid_v.at[pl.ds(cb * NEG + i * IDXBLK, IDXBLK)]],
                nrow2.at[pl.ds(i * IDXBLK, IDXBLK)], gsem))
        for cp in copies:
            cp.wait()

        for grp in range(GC):
            crows = lanes + grp * L
            acc_p = jnp.zeros((L,), jnp.float32)
            for db in range(D // L):
                ct = [plsc.load_gather(crow, [crows, cols[db * L + k]])
                      for k in range(L)]
                for k in range(L):
                    x = plsc.load_gather(prow, [crows, cols[db * L + k]])
                    acc_p = acc_p + ct[k] * x

                def jbody(j, _, db=db, ct=ct, crows=crows, grp=grp, g=g):
                    nr = crows * NEG + j
                    t = jnp.zeros((L,), jnp.float32)
                    for k in range(L):
                        x = plsc.load_gather(nrow, [nr, cols[db * L + k]])
                        t = t + ct[k] * x
                    off = g * (C * NEG) + j * C + grp * L
                    if db == 0:
                        nsc[pl.ds(off, L)] = t
                    else:
                        nsc[pl.ds(off, L)] = nsc[pl.ds(off, L)] + t
                    return 0

                lax.fori_loop(0, NEG, jbody, 0)
            psc[pl.ds(g * C + grp * L, L)] = acc_p
        return carry

    lax.fori_loop(0, NCHUNK, chunk_body, 0)

    pltpu.sync_copy(psc, pos_out.at[pl.ds(base, BW)])
    pltpu.sync_copy(nsc, neg_out.at[pl.ds(base * NEG, BW * NEG)])


def _sc_score(cid, pid, nid, cw, xw):
    mesh = plsc.VectorSubcoreMesh(core_axis_name="c", subcore_axis_name="s")
    f = functools.partial(
        pl.kernel,
        mesh=mesh,
        compiler_params=pltpu.CompilerParams(
            needs_layout_passes=False, use_tc_tiling_on_sc=False),
        out_type=[
            jax.ShapeDtypeStruct((B,), jnp.float32),
            jax.ShapeDtypeStruct((B * NEG,), jnp.float32),
        ],
        scratch_types=[
            pltpu.VMEM((BW,), jnp.int32),
            pltpu.VMEM((BW,), jnp.int32),
            pltpu.VMEM((BW * NEG,), jnp.int32),
            pltpu.VMEM((C, D), jnp.float32),
            pltpu.VMEM((C, D), jnp.float32),
            pltpu.VMEM((C * NEG, D), jnp.float32),
            pltpu.VMEM((BW,), jnp.float32),
            pltpu.VMEM((BW * NEG,), jnp.float32),
            pltpu.SemaphoreType.DMA,
        ],
    )(_sc_score_body)
    return f(cid, pid, nid, cw, xw)


def _loss_body(pos_ref, neg_ref, out_ref):
    p = pos_ref[...]
    n = neg_ref[...]
    lsp = jnp.minimum(p, 0.0) - jnp.log1p(jnp.exp(-jnp.abs(p)))
    lsn = jnp.minimum(-n, 0.0) - jnp.log1p(jnp.exp(-jnp.abs(n)))
    out_ref[0, 0] = -(jnp.sum(lsp) + jnp.sum(lsn)) / B


def _loss(pos2d, neg2d):
    return pl.pallas_call(
        _loss_body,
        out_shape=jax.ShapeDtypeStruct((1, 1), jnp.float32),
        in_specs=[
            pl.BlockSpec(memory_space=pltpu.VMEM),
            pl.BlockSpec(memory_space=pltpu.VMEM),
        ],
        out_specs=pl.BlockSpec(memory_space=pltpu.SMEM),
    )(pos2d, neg2d)


def kernel(center_id, context_ids, negative_ids, center_w, context_w):
    cid = center_id.astype(jnp.int32)
    pid = context_ids.astype(jnp.int32)
    nid = negative_ids.astype(jnp.int32).reshape(B * NEG)
    pos_sc, neg_sc = _sc_score(cid, pid, nid, center_w, context_w)
    out = _loss(pos_sc.reshape(B // 128, 128), neg_sc.reshape(B * NEG // 128, 128))
    return out[0, 0]


# speedup vs baseline: 1.0230x; 1.0230x over previous
import functools

import jax
import jax.numpy as jnp
from jax import lax
from jax.experimental import pallas as pl
from jax.experimental.pallas import tpu as pltpu
from jax.experimental.pallas import tpu_sc as plsc

B = 16384
V = 1000000
D = 64
NEG = 20

NC = 2
NS = 16
L = 16
NW = NC * NS
BW = B // NW
C = 32
NCHUNK = BW // C
GC = C // L
IDXBLK = 128


def _sc_score_body(cid_hbm, pid_hbm, nid_hbm, cw_hbm, xw_hbm,
                   pos_out, neg_out,
                   cid_v, pid_v, nid_v, crow0, crow1, prow0, prow1,
                   nrow0, nrow1, psc, nsc, gsem0, gsem1):
    c = lax.axis_index("c")
    s = lax.axis_index("s")
    wid = s * NC + c
    base = wid * BW
    crow = (crow0, crow1)
    prow = (prow0, prow1)
    nrow = (nrow0, nrow1)
    gsem = (gsem0, gsem1)

    pltpu.sync_copy(cid_hbm.at[pl.ds(base, BW)], cid_v)
    pltpu.sync_copy(pid_hbm.at[pl.ds(base, BW)], pid_v)
    pltpu.sync_copy(nid_hbm.at[pl.ds(base * NEG, BW * NEG)], nid_v)

    lanes = lax.iota(jnp.int32, L)
    cols = [jnp.full((L,), d, jnp.int32) for d in range(D)]

    def chunk_copies(g, sub, make):
        cb = g * C
        f = pltpu.make_async_copy if make else pltpu.async_copy
        out = [
            f(cw_hbm.at[cid_v.at[pl.ds(cb, C)]], crow[sub], gsem[sub]),
            f(xw_hbm.at[pid_v.at[pl.ds(cb, C)]], prow[sub], gsem[sub]),
        ]
        for i in range(C * NEG // IDXBLK):
            out.append(f(
                xw_hbm.at[nid_v.at[pl.ds(cb * NEG + i * IDXBLK, IDXBLK)]],
                nrow[sub].at[pl.ds(i * IDXBLK, IDXBLK)], gsem[sub]))
        return out

    def compute_chunk(g, sub):
        for grp in range(GC):
            crows = lanes + grp * L
            acc_p = jnp.zeros((L,), jnp.float32)
            for db in range(D // L):
                ct = [plsc.load_gather(crow[sub], [crows, cols[db * L + k]])
                      for k in range(L)]
                for k in range(L):
                    x = plsc.load_gather(prow[sub], [crows, cols[db * L + k]])
                    acc_p = acc_p + ct[k] * x

                def jbody(j, _, db=db, ct=ct, crows=crows, grp=grp, g=g,
                          sub=sub):
                    nr = crows * NEG + j
                    t = jnp.zeros((L,), jnp.float32)
                    for k in range(L):
                        x = plsc.load_gather(nrow[sub], [nr, cols[db * L + k]])
                        t = t + ct[k] * x
                    off = g * (C * NEG) + j * C + grp * L
                    if db == 0:
                        nsc[pl.ds(off, L)] = t
                    else:
                        nsc[pl.ds(off, L)] = nsc[pl.ds(off, L)] + t
                    return 0

                lax.fori_loop(0, NEG, jbody, 0)
            psc[pl.ds(g * C + grp * L, L)] = acc_p

    chunk_copies(0, 0, make=False)
    chunk_copies(1, 1, make=False)

    def pair_body(p, carry):
        for sub in range(2):
            g = p * 2 + sub
            for cp in chunk_copies(g, sub, make=True):
                cp.wait()
            compute_chunk(g, sub)

            @pl.when(g + 2 < NCHUNK)
            def _():
                chunk_copies(g + 2, sub, make=False)
        return carry

    lax.fori_loop(0, NCHUNK // 2, pair_body, 0)

    pltpu.sync_copy(psc, pos_out.at[pl.ds(base, BW)])
    pltpu.sync_copy(nsc, neg_out.at[pl.ds(base * NEG, BW * NEG)])


def _sc_score(cid, pid, nid, cw, xw):
    mesh = plsc.VectorSubcoreMesh(core_axis_name="c", subcore_axis_name="s")
    f = functools.partial(
        pl.kernel,
        mesh=mesh,
        compiler_params=pltpu.CompilerParams(
            needs_layout_passes=False, use_tc_tiling_on_sc=False),
        out_type=[
            jax.ShapeDtypeStruct((B,), jnp.float32),
            jax.ShapeDtypeStruct((B * NEG,), jnp.float32),
        ],
        scratch_types=[
            pltpu.VMEM((BW,), jnp.int32),
            pltpu.VMEM((BW,), jnp.int32),
            pltpu.VMEM((BW * NEG,), jnp.int32),
            pltpu.VMEM((C, D), jnp.float32),
            pltpu.VMEM((C, D), jnp.float32),
            pltpu.VMEM((C, D), jnp.float32),
            pltpu.VMEM((C, D), jnp.float32),
            pltpu.VMEM((C * NEG, D), jnp.float32),
            pltpu.VMEM((C * NEG, D), jnp.float32),
            pltpu.VMEM((BW,), jnp.float32),
            pltpu.VMEM((BW * NEG,), jnp.float32),
            pltpu.SemaphoreType.DMA,
            pltpu.SemaphoreType.DMA,
        ],
    )(_sc_score_body)
    return f(cid, pid, nid, cw, xw)


def _loss_body(pos_ref, neg_ref, out_ref):
    p = pos_ref[...]
    n = neg_ref[...]
    lsp = jnp.minimum(p, 0.0) - jnp.log1p(jnp.exp(-jnp.abs(p)))
    lsn = jnp.minimum(-n, 0.0) - jnp.log1p(jnp.exp(-jnp.abs(n)))
    out_ref[0, 0] = -(jnp.sum(lsp) + jnp.sum(lsn)) / B


def _loss(pos2d, neg2d):
    return pl.pallas_call(
        _loss_body,
        out_shape=jax.ShapeDtypeStruct((1, 1), jnp.float32),
        in_specs=[
            pl.BlockSpec(memory_space=pltpu.VMEM),
            pl.BlockSpec(memory_space=pltpu.VMEM),
        ],
        out_specs=pl.BlockSpec(memory_space=pltpu.SMEM),
    )(pos2d, neg2d)


def kernel(center_id, context_ids, negative_ids, center_w, context_w):
    cid = center_id.astype(jnp.int32)
    pid = context_ids.astype(jnp.int32)
    nid = negative_ids.astype(jnp.int32).reshape(B * NEG)
    pos_sc, neg_sc = _sc_score(cid, pid, nid, center_w, context_w)
    out = _loss(pos_sc.reshape(B // 128, 128), neg_sc.reshape(B * NEG // 128, 128))
    return out[0, 0]


# speedup vs baseline: 1.0236x; 1.0006x over previous
import functools

import jax
import jax.numpy as jnp
from jax import lax
from jax.experimental import pallas as pl
from jax.experimental.pallas import tpu as pltpu
from jax.experimental.pallas import tpu_sc as plsc

B = 16384
V = 1000000
D = 64
NEG = 20

NC = 2
NS = 16
L = 16
NW = NC * NS
BW = B // NW
C = 32
NCHUNK = BW // C
GC = C // L
H = 32


def _sc_score_body(cid_hbm, pid_hbm, nid_hbm, cw_hbm, xw_hbm,
                   pos_out, neg_out,
                   cid_v, pid_v, nid_v, crow0, crow1, prow0, prow1,
                   nrow0, nrow1, psc, nsc, gsem0, gsem1):
    c = lax.axis_index("c")
    s = lax.axis_index("s")
    wid = s * NC + c
    base = wid * BW
    crow = (crow0, crow1)
    prow = (prow0, prow1)
    nrow = (nrow0, nrow1)
    gsem = (gsem0, gsem1)

    pltpu.sync_copy(cid_hbm.at[pl.ds(base, BW)], cid_v)
    pltpu.sync_copy(pid_hbm.at[pl.ds(base, BW)], pid_v)
    for j in range(NEG):
        pltpu.sync_copy(nid_hbm.at[j, pl.ds(base, BW)], nid_v.at[j])

    lanes = lax.iota(jnp.int32, L)
    cols = [jnp.full((L,), d, jnp.int32) for d in range(D)]

    def chunk_copies(g, sub, make):
        cb = g * C
        f = pltpu.make_async_copy if make else pltpu.async_copy
        out = [
            f(cw_hbm.at[cid_v.at[pl.ds(cb, C)]], crow[sub], gsem[sub]),
            f(xw_hbm.at[pid_v.at[pl.ds(cb, C)]], prow[sub], gsem[sub]),
        ]
        for j in range(NEG):
            out.append(f(
                xw_hbm.at[nid_v.at[j, pl.ds(cb, C)]],
                nrow[sub].at[pl.ds(j * C, C)], gsem[sub]))
        return out

    def compute_chunk(g, sub):
        for grp in range(GC):
            crows = lanes + grp * L
            ap = [jnp.zeros((L,), jnp.float32) for _ in range(4)]
            for db in range(D // H):
                ct = [plsc.load_gather(crow[sub], [crows, cols[db * H + k]])
                      for k in range(H)]
                for k in range(H):
                    x = plsc.load_gather(prow[sub], [crows, cols[db * H + k]])
                    ap[k % 4] = ap[k % 4] + ct[k] * x

                def jbody(j, _, db=db, ct=ct, crows=crows, grp=grp, g=g,
                          sub=sub):
                    nr = crows + j * C
                    t = [jnp.zeros((L,), jnp.float32) for _ in range(4)]
                    for k in range(H):
                        x = plsc.load_gather(nrow[sub],
                                             [nr, cols[db * H + k]])
                        t[k % 4] = t[k % 4] + ct[k] * x
                    tt = (t[0] + t[1]) + (t[2] + t[3])
                    off = g * (C * NEG) + j * C + grp * L
                    if db == 0:
                        nsc[pl.ds(off, L)] = tt
                    else:
                        nsc[pl.ds(off, L)] = nsc[pl.ds(off, L)] + tt
                    return 0

                lax.fori_loop(0, NEG, jbody, 0)
            psc[pl.ds(g * C + grp * L, L)] = (ap[0] + ap[1]) + (ap[2] + ap[3])

    chunk_copies(0, 0, make=False)
    chunk_copies(1, 1, make=False)

    def pair_body(p, carry):
        for sub in range(2):
            g = p * 2 + sub
            for cp in chunk_copies(g, sub, make=True):
                cp.wait()
            compute_chunk(g, sub)

            @pl.when(g + 2 < NCHUNK)
            def _():
                chunk_copies(g + 2, sub, make=False)
        return carry

    lax.fori_loop(0, NCHUNK // 2, pair_body, 0)

    pltpu.sync_copy(psc, pos_out.at[pl.ds(base, BW)])
    pltpu.sync_copy(nsc, neg_out.at[pl.ds(base * NEG, BW * NEG)])


def _sc_score(cid, pid, nid_t, cw, xw):
    mesh = plsc.VectorSubcoreMesh(core_axis_name="c", subcore_axis_name="s")
    f = functools.partial(
        pl.kernel,
        mesh=mesh,
        compiler_params=pltpu.CompilerParams(
            needs_layout_passes=False, use_tc_tiling_on_sc=False),
        out_type=[
            jax.ShapeDtypeStruct((B,), jnp.float32),
            jax.ShapeDtypeStruct((B * NEG,), jnp.float32),
        ],
        scratch_types=[
            pltpu.VMEM((BW,), jnp.int32),
            pltpu.VMEM((BW,), jnp.int32),
            pltpu.VMEM((NEG, BW), jnp.int32),
            pltpu.VMEM((C, D), jnp.float32),
            pltpu.VMEM((C, D), jnp.float32),
            pltpu.VMEM((C, D), jnp.float32),
            pltpu.VMEM((C, D), jnp.float32),
            pltpu.VMEM((C * NEG, D), jnp.float32),
            pltpu.VMEM((C * NEG, D), jnp.float32),
            pltpu.VMEM((BW,), jnp.float32),
            pltpu.VMEM((BW * NEG,), jnp.float32),
            pltpu.SemaphoreType.DMA,
            pltpu.SemaphoreType.DMA,
        ],
    )(_sc_score_body)
    return f(cid, pid, nid_t, cw, xw)


def _loss_body(pos_ref, neg_ref, out_ref):
    p = pos_ref[...]
    n = neg_ref[...]
    lsp = jnp.minimum(p, 0.0) - jnp.log1p(jnp.exp(-jnp.abs(p)))
    lsn = jnp.minimum(-n, 0.0) - jnp.log1p(jnp.exp(-jnp.abs(n)))
    out_ref[0, 0] = -(jnp.sum(lsp) + jnp.sum(lsn)) / B


def _loss(pos2d, neg2d):
    return pl.pallas_call(
        _loss_body,
        out_shape=jax.ShapeDtypeStruct((1, 1), jnp.float32),
        in_specs=[
            pl.BlockSpec(memory_space=pltpu.VMEM),
            pl.BlockSpec(memory_space=pltpu.VMEM),
        ],
        out_specs=pl.BlockSpec(memory_space=pltpu.SMEM),
    )(pos2d, neg2d)


def kernel(center_id, context_ids, negative_ids, center_w, context_w):
    cid = center_id.astype(jnp.int32)
    pid = context_ids.astype(jnp.int32)
    nid_t = negative_ids.astype(jnp.int32).T
    pos_sc, neg_sc = _sc_score(cid, pid, nid_t, center_w, context_w)
    out = _loss(pos_sc.reshape(B // 128, 128), neg_sc.reshape(B * NEG // 128, 128))
    return out[0, 0]
